# baseline (speedup 1.0000x reference)
import functools

import jax
import jax.numpy as jnp
from jax import lax
from jax.experimental import pallas as pl
from jax.experimental.pallas import tpu as pltpu
from jax.experimental.pallas import tpu_sc as plsc

NUM_NODES = 100000
NUM_REL = 500
NUM_ANC = 20000
ANCS = 20
RELCTX = 12
NTOK = ANCS + RELCTX
DIM = 64
B = 16384
REL_OFF = NUM_ANC + 1
IN_DIM = NTOK * DIM

NSLICE = 2
BS = B // NSLICE

_info = plsc.get_sparse_core_info()
NC, NS = _info.num_cores, _info.num_subcores
NW = NC * NS

NODES_PER_W = B // NW
CHUNK = 512
NCHUNKS = BS // NW * NTOK // CHUNK

_mesh = plsc.VectorSubcoreMesh(core_axis_name="c", subcore_axis_name="s")
_sc_params = pltpu.CompilerParams(use_tc_tiling_on_sc=False)


@functools.partial(
    pl.kernel,
    mesh=_mesh,
    compiler_params=_sc_params,
    out_type=jax.ShapeDtypeStruct((NW, NODES_PER_W, NTOK), jnp.int32),
    scratch_types=[
        pltpu.VMEM((NODES_PER_W,), jnp.int32),
        pltpu.VMEM((NODES_PER_W, NTOK), jnp.int32),
        pltpu.SemaphoreType.DMA,
    ],
)
def _gather_tokens(idx_hbm, hashes_hbm, tok_hbm, idx_v, tok_v, sem):
    wid = lax.axis_index("s") * NC + lax.axis_index("c")
    base = wid * NODES_PER_W
    pltpu.sync_copy(idx_hbm.at[pl.ds(base, NODES_PER_W)], idx_v)
    pltpu.async_copy(hashes_hbm.at[idx_v], tok_v, sem).wait()

    lanes = lax.iota(jnp.int32, 16)
    off = jnp.where(lanes >= ANCS - 16, REL_OFF, 0)

    def body(r, _):
        tok_v[r, pl.ds(16, 16)] = tok_v[r, pl.ds(16, 16)] + off
        return 0

    lax.fori_loop(0, NODES_PER_W, body, 0)
    pltpu.sync_copy(tok_v, tok_hbm.at[wid])


@functools.partial(
    pl.kernel,
    mesh=_mesh,
    compiler_params=_sc_params,
    out_type=jax.ShapeDtypeStruct((NW, NCHUNKS, CHUNK, DIM), jnp.bfloat16),
    scratch_types=[
        pltpu.VMEM((NCHUNKS, CHUNK), jnp.int32),
        pltpu.VMEM((2, CHUNK, DIM), jnp.bfloat16),
        pltpu.SemaphoreType.DMA((2,)),
    ],
)
def _gather_embs(emb_hbm, tok_hbm, out_hbm, idx_v, buf, sem):
    wid = lax.axis_index("s") * NC + lax.axis_index("c")
    pltpu.sync_copy(tok_hbm.at[wid], idx_v)

    pltpu.async_copy(emb_hbm.at[idx_v.at[0]], buf.at[0], sem.at[0])

    def body(c, _):
        nxt = c + 1

        @pl.when(nxt < NCHUNKS)
        def _():
            pltpu.make_async_copy(
                emb_hbm.at[idx_v.at[nxt]], buf.at[nxt % 2], sem.at[nxt % 2]
            ).start()

        pltpu.make_async_copy(
            emb_hbm.at[idx_v.at[c]], buf.at[c % 2], sem.at[c % 2]
        ).wait()
        pltpu.sync_copy(buf.at[c % 2], out_hbm.at[wid, c])
        return 0

    lax.fori_loop(0, NCHUNKS, body, 0)


ROWS_BLK = 512


def _mlp_body(x_ref, w1_ref, b1_ref, w2_ref, b2_ref, out_ref):
    x = x_ref[...].reshape(ROWS_BLK, IN_DIM)
    h = jnp.dot(x, w1_ref[...], preferred_element_type=jnp.float32)
    h = jnp.maximum(h + b1_ref[...], 0.0)
    out_ref[...] = (
        jnp.dot(h, w2_ref[...], preferred_element_type=jnp.float32)
        + b2_ref[...]
    )


def _mlp(x_flat, w1, b1, w2, b2):
    grid = (BS // ROWS_BLK,)
    return pl.pallas_call(
        _mlp_body,
        grid=grid,
        in_specs=[
            pl.BlockSpec((ROWS_BLK * IN_DIM,), lambda i: (i,)),
            pl.BlockSpec((IN_DIM, 2 * DIM), lambda i: (0, 0)),
            pl.BlockSpec((1, 2 * DIM), lambda i: (0, 0)),
            pl.BlockSpec((2 * DIM, DIM), lambda i: (0, 0)),
            pl.BlockSpec((1, DIM), lambda i: (0, 0)),
        ],
        out_specs=pl.BlockSpec((ROWS_BLK, DIM), lambda i: (i, 0)),
        out_shape=jax.ShapeDtypeStruct((BS, DIM), jnp.float32),
    )(x_flat, w1, b1, w2, b2)


def kernel(indices, anchor_hashes, node_hashes, relations, anchor_emb,
           W1, b1, W2, b2):
    hashes = jnp.concatenate([anchor_hashes, node_hashes], axis=1)
    emb = jnp.concatenate([anchor_emb, relations],
                          axis=0).astype(jnp.bfloat16)
    tok = _gather_tokens(indices, hashes)
    w1 = W1.astype(jnp.bfloat16)
    b1r = b1.reshape(1, -1)
    b2r = b2.reshape(1, -1)
    wps = NW // NSLICE
    outs = []
    for s in range(NSLICE):
        tok_s = tok[s * wps:(s + 1) * wps].reshape(NW, NCHUNKS, CHUNK)
        rows = _gather_embs(emb, tok_s)
        x_flat = rows.reshape(BS * IN_DIM)
        outs.append(_mlp(x_flat, w1, b1r, W2, b2r))
    return jnp.concatenate(outs, axis=0)


# speedup vs baseline: 1.0175x; 1.0175x over previous
import functools

import jax
import jax.numpy as jnp
from jax import lax
from jax.experimental import pallas as pl
from jax.experimental.pallas import tpu as pltpu
from jax.experimental.pallas import tpu_sc as plsc

NUM_NODES = 100000
NUM_REL = 500
NUM_ANC = 20000
ANCS = 20
RELCTX = 12
NTOK = ANCS + RELCTX
DIM = 64
B = 16384
REL_OFF = NUM_ANC + 1
IN_DIM = NTOK * DIM

NSLICE = 2
BS = B // NSLICE

_info = plsc.get_sparse_core_info()
NC, NS = _info.num_cores, _info.num_subcores
NW = NC * NS

NODES_PER_W = B // NW
CHUNK = 256
NCHUNKS = BS // NW * NTOK // CHUNK

_mesh = plsc.VectorSubcoreMesh(core_axis_name="c", subcore_axis_name="s")
_sc_params = pltpu.CompilerParams(use_tc_tiling_on_sc=False)


@functools.partial(
    pl.kernel,
    mesh=_mesh,
    compiler_params=_sc_params,
    out_type=jax.ShapeDtypeStruct((NW, NODES_PER_W, NTOK), jnp.int32),
    scratch_types=[
        pltpu.VMEM((NODES_PER_W,), jnp.int32),
        pltpu.VMEM((NODES_PER_W, NTOK), jnp.int32),
        pltpu.SemaphoreType.DMA,
    ],
)
def _gather_tokens(idx_hbm, hashes_hbm, tok_hbm, idx_v, tok_v, sem):
    wid = lax.axis_index("s") * NC + lax.axis_index("c")
    base = wid * NODES_PER_W
    pltpu.sync_copy(idx_hbm.at[pl.ds(base, NODES_PER_W)], idx_v)
    pltpu.async_copy(hashes_hbm.at[idx_v], tok_v, sem).wait()

    lanes = lax.iota(jnp.int32, 16)
    off = jnp.where(lanes >= ANCS - 16, REL_OFF, 0)

    def body(r, _):
        tok_v[r, pl.ds(16, 16)] = tok_v[r, pl.ds(16, 16)] + off
        return 0

    lax.fori_loop(0, NODES_PER_W, body, 0)
    pltpu.sync_copy(tok_v, tok_hbm.at[wid])


@functools.partial(
    pl.kernel,
    mesh=_mesh,
    compiler_params=_sc_params,
    out_type=jax.ShapeDtypeStruct((NW, NCHUNKS, CHUNK, DIM), jnp.bfloat16),
    scratch_types=[
        pltpu.VMEM((NCHUNKS, CHUNK), jnp.int32),
        pltpu.VMEM((2, CHUNK, DIM), jnp.bfloat16),
        pltpu.SemaphoreType.DMA((2,)),
    ],
)
def _gather_embs(emb_hbm, tok_hbm, out_hbm, idx_v, buf, sem):
    wid = lax.axis_index("s") * NC + lax.axis_index("c")
    pltpu.sync_copy(tok_hbm.at[wid], idx_v)

    pltpu.async_copy(emb_hbm.at[idx_v.at[0]], buf.at[0], sem.at[0])

    def body(c, _):
        nxt = c + 1

        @pl.when(nxt < NCHUNKS)
        def _():
            pltpu.make_async_copy(
                emb_hbm.at[idx_v.at[nxt]], buf.at[nxt % 2], sem.at[nxt % 2]
            ).start()

        pltpu.make_async_copy(
            emb_hbm.at[idx_v.at[c]], buf.at[c % 2], sem.at[c % 2]
        ).wait()
        pltpu.sync_copy(buf.at[c % 2], out_hbm.at[wid, c])
        return 0

    lax.fori_loop(0, NCHUNKS, body, 0)


ROWS_BLK = 1024


def _mlp_body(x_ref, w1_ref, b1_ref, w2_ref, b2_ref, out_ref):
    x = x_ref[...].reshape(ROWS_BLK, IN_DIM)
    h = jnp.dot(x, w1_ref[...], preferred_element_type=jnp.float32)
    h = jnp.maximum(h + b1_ref[...], 0.0)
    out_ref[...] = (
        jnp.dot(h, w2_ref[...], preferred_element_type=jnp.float32)
        + b2_ref[...]
    )


def _mlp(x_flat, w1, b1, w2, b2):
    grid = (BS // ROWS_BLK,)
    return pl.pallas_call(
        _mlp_body,
        grid=grid,
        in_specs=[
            pl.BlockSpec((ROWS_BLK * IN_DIM,), lambda i: (i,)),
            pl.BlockSpec((IN_DIM, 2 * DIM), lambda i: (0, 0)),
            pl.BlockSpec((1, 2 * DIM), lambda i: (0, 0)),
            pl.BlockSpec((2 * DIM, DIM), lambda i: (0, 0)),
            pl.BlockSpec((1, DIM), lambda i: (0, 0)),
        ],
        out_specs=pl.BlockSpec((ROWS_BLK, DIM), lambda i: (i, 0)),
        out_shape=jax.ShapeDtypeStruct((BS, DIM), jnp.float32),
    )(x_flat, w1, b1, w2, b2)


def kernel(indices, anchor_hashes, node_hashes, relations, anchor_emb,
           W1, b1, W2, b2):
    hashes = jnp.concatenate([anchor_hashes, node_hashes], axis=1)
    emb = jnp.concatenate([anchor_emb, relations],
                          axis=0).astype(jnp.bfloat16)
    tok = _gather_tokens(indices, hashes)
    w1 = W1.astype(jnp.bfloat16)
    b1r = b1.reshape(1, -1)
    b2r = b2.reshape(1, -1)
    wps = NW // NSLICE
    outs = []
    for s in range(NSLICE):
        tok_s = tok[s * wps:(s + 1) * wps].reshape(NW, NCHUNKS, CHUNK)
        rows = _gather_embs(emb, tok_s)
        x_flat = rows.reshape(BS * IN_DIM)
        outs.append(_mlp(x_flat, w1, b1r, W2, b2r))
    return jnp.concatenate(outs, axis=0)


# speedup vs baseline: 1.0181x; 1.0006x over previous
import functools

import jax
import jax.numpy as jnp
from jax import lax
from jax.experimental import pallas as pl
from jax.experimental.pallas import tpu as pltpu
from jax.experimental.pallas import tpu_sc as plsc

NUM_NODES = 100000
NUM_REL = 500
NUM_ANC = 20000
ANCS = 20
RELCTX = 12
NTOK = ANCS + RELCTX
DIM = 64
B = 16384
REL_OFF = NUM_ANC + 1
IN_DIM = NTOK * DIM

NSLICE = 2
BS = B // NSLICE

_info = plsc.get_sparse_core_info()
NC, NS = _info.num_cores, _info.num_subcores
NW = NC * NS

NODES_PER_W = B // NW
CHUNK = 256
NCHUNKS = BS // NW * NTOK // CHUNK

_mesh = plsc.VectorSubcoreMesh(core_axis_name="c", subcore_axis_name="s")
_sc_params = pltpu.CompilerParams(use_tc_tiling_on_sc=False)


@functools.partial(
    pl.kernel,
    mesh=_mesh,
    compiler_params=_sc_params,
    out_type=jax.ShapeDtypeStruct((NW, NODES_PER_W, NTOK), jnp.int32),
    scratch_types=[
        pltpu.VMEM((NODES_PER_W,), jnp.int32),
        pltpu.VMEM((NODES_PER_W, NTOK), jnp.int32),
        pltpu.SemaphoreType.DMA,
    ],
)
def _gather_tokens(idx_hbm, hashes_hbm, tok_hbm, idx_v, tok_v, sem):
    wid = lax.axis_index("s") * NC + lax.axis_index("c")
    base = wid * NODES_PER_W
    pltpu.sync_copy(idx_hbm.at[pl.ds(base, NODES_PER_W)], idx_v)
    pltpu.async_copy(hashes_hbm.at[idx_v], tok_v, sem).wait()

    lanes = lax.iota(jnp.int32, 16)
    off = jnp.where(lanes >= ANCS - 16, REL_OFF, 0)

    def body(r, _):
        tok_v[r, pl.ds(16, 16)] = tok_v[r, pl.ds(16, 16)] + off
        return 0

    lax.fori_loop(0, NODES_PER_W, body, 0)
    pltpu.sync_copy(tok_v, tok_hbm.at[wid])


@functools.partial(
    pl.kernel,
    mesh=_mesh,
    compiler_params=_sc_params,
    out_type=jax.ShapeDtypeStruct((NW, NCHUNKS, CHUNK, DIM), jnp.bfloat16),
    scratch_types=[
        pltpu.VMEM((NCHUNKS, CHUNK), jnp.int32),
        pltpu.VMEM((4, CHUNK, DIM), jnp.bfloat16),
        pltpu.SemaphoreType.DMA((4,)),
        pltpu.SemaphoreType.DMA((4,)),
    ],
)
def _gather_embs(emb_hbm, tok_hbm, out_hbm, idx_v, buf, gsem, wsem):
    wid = lax.axis_index("s") * NC + lax.axis_index("c")
    pltpu.sync_copy(tok_hbm.at[wid], idx_v)

    def gather(c):
        return pltpu.make_async_copy(
            emb_hbm.at[idx_v.at[c]], buf.at[c % 4], gsem.at[c % 4])

    def write(c):
        return pltpu.make_async_copy(
            buf.at[c % 4], out_hbm.at[wid, c], wsem.at[c % 4])

    gather(0).start()

    def body(c, _):
        nxt = c + 1

        @pl.when(nxt < NCHUNKS)
        def _():
            @pl.when(nxt >= 4)
            def _():
                write(nxt - 4).wait()

            gather(nxt).start()

        gather(c).wait()
        write(c).start()
        return 0

    lax.fori_loop(0, NCHUNKS, body, 0)

    def drain(c, _):
        write(c).wait()
        return 0

    lax.fori_loop(max(0, NCHUNKS - 4), NCHUNKS, drain, 0)


ROWS_BLK = 1024


def _mlp_body(x_ref, w1_ref, b1_ref, w2_ref, b2_ref, out_ref):
    x = x_ref[...].reshape(ROWS_BLK, IN_DIM)
    h = jnp.dot(x, w1_ref[...], preferred_element_type=jnp.float32)
    h = jnp.maximum(h + b1_ref[...], 0.0)
    out_ref[...] = (
        jnp.dot(h, w2_ref[...], preferred_element_type=jnp.float32)
        + b2_ref[...]
    )


def _mlp(x_flat, w1, b1, w2, b2):
    grid = (BS // ROWS_BLK,)
    return pl.pallas_call(
        _mlp_body,
        grid=grid,
        in_specs=[
            pl.BlockSpec((ROWS_BLK * IN_DIM,), lambda i: (i,)),
            pl.BlockSpec((IN_DIM, 2 * DIM), lambda i: (0, 0)),
            pl.BlockSpec((1, 2 * DIM), lambda i: (0, 0)),
            pl.BlockSpec((2 * DIM, DIM), lambda i: (0, 0)),
            pl.BlockSpec((1, DIM), lambda i: (0, 0)),
        ],
        out_specs=pl.BlockSpec((ROWS_BLK, DIM), lambda i: (i, 0)),
        out_shape=jax.ShapeDtypeStruct((BS, DIM), jnp.float32),
    )(x_flat, w1, b1, w2, b2)


def kernel(indices, anchor_hashes, node_hashes, relations, anchor_emb,
           W1, b1, W2, b2):
    hashes = jnp.concatenate([anchor_hashes, node_hashes], axis=1)
    emb = jnp.concatenate([anchor_emb, relations],
                          axis=0).astype(jnp.bfloat16)
    tok = _gather_tokens(indices, hashes)
    w1 = W1.astype(jnp.bfloat16)
    b1r = b1.reshape(1, -1)
    b2r = b2.reshape(1, -1)
    wps = NW // NSLICE
    outs = []
    for s in range(NSLICE):
        tok_s = tok[s * wps:(s + 1) * wps].reshape(NW, NCHUNKS, CHUNK)
        rows = _gather_embs(emb, tok_s)
        x_flat = rows.reshape(BS * IN_DIM)
        outs.append(_mlp(x_flat, w1, b1r, W2, b2r))
    return jnp.concatenate(outs, axis=0)
